# baseline (speedup 1.0000x reference)
import jax
import jax.numpy as jnp
from jax import lax
from jax.experimental import pallas as pl
from jax.experimental.pallas import tpu as pltpu
from jax.experimental.pallas import tpu_sc as plsc

B = 4
N = 12500
NP = 12800
E = 200000
EPB = 204800
H = 64
H2 = 32
NS = 30
NC = 300
NR = 16

BN = 1280
NB = NP // BN

NCORES = 2
NSUB = 16
C = 80
CHUNKS_SID = 160
K0 = 125
K1 = CHUNKS_SID - K0
ROWS_PER_SUB = NP // NSUB
DUMMY_DST = 12600

RBUF = 10
DEPTH = 5

_F32 = jnp.float32
_BF16 = jnp.bfloat16


def _mlp_einsum_body(cls_ref, st_ref, cemb_ref, swt_ref, sb_ref,
                     w1t_ref, b1_ref, w2t_ref, b2_ref, wcat_ref, out_ref):
    cls = cls_ref[:, :].astype(jnp.int32)
    iota = lax.broadcasted_iota(jnp.int32, (BN, NC), 1)
    onehot = (iota == cls).astype(_F32)
    cn = jnp.dot(onehot, cemb_ref[...], preferred_element_type=_F32)
    se = jnp.dot(st_ref[...], swt_ref[...], preferred_element_type=_F32)
    se = se + sb_ref[...]
    h = jax.nn.relu(jnp.concatenate([cn, se], axis=1))
    h = jax.nn.relu(jnp.dot(h, w1t_ref[...], preferred_element_type=_F32)
                    + b1_ref[...])
    h = jax.nn.relu(jnp.dot(h, w2t_ref[...], preferred_element_type=_F32)
                    + b2_ref[...])
    hw = jnp.dot(h, wcat_ref[...],
                 preferred_element_type=_F32).astype(_BF16)
    for r in range(NR):
        out_ref[r * BN:(r + 1) * BN, :] = hw[:, r * H:(r + 1) * H]


def _mlp_einsum(cls_b, states_b, class_emb, swt, sb, w1t, b1, w2t, b2, wcat):
    full = lambda shape: pl.BlockSpec(shape, lambda i: (0,) * len(shape))
    return pl.pallas_call(
        _mlp_einsum_body,
        grid=(NB,),
        in_specs=[
            pl.BlockSpec((BN, 1), lambda i: (i, 0)),
            pl.BlockSpec((BN, NS), lambda i: (i, 0)),
            full((NC, H2)), full((NS, H2)), full((1, H2)),
            full((H, H)), full((1, H)), full((H, H)), full((1, H)),
            full((H, NR * H)),
        ],
        out_specs=pl.BlockSpec((NR * BN, H), lambda i: (i, 0)),
        out_shape=jax.ShapeDtypeStruct((NP * NR, H), _BF16),
    )(cls_b, states_b, class_emb, swt, sb, w1t, b1, w2t, b2, wcat)


def _combine_einsum_body(p_ref, wcat_ref, out_ref):
    h = jax.nn.relu(p_ref[0].astype(_F32) + p_ref[1].astype(_F32))
    hw = jnp.dot(h, wcat_ref[...],
                 preferred_element_type=_F32).astype(_BF16)
    for r in range(NR):
        out_ref[r * BN:(r + 1) * BN, :] = hw[:, r * H:(r + 1) * H]


def _combine_einsum(p, wcat):
    return pl.pallas_call(
        _combine_einsum_body,
        grid=(NB,),
        in_specs=[
            pl.BlockSpec((2, BN, H), lambda i: (0, i, 0)),
            pl.BlockSpec((H, NR * H), lambda i: (0, 0)),
        ],
        out_specs=pl.BlockSpec((NR * BN, H), lambda i: (i, 0)),
        out_shape=jax.ShapeDtypeStruct((NP * NR, H), _BF16),
    )(p, wcat)


def _combine_body(p_ref, out_ref):
    out_ref[...] = jax.nn.relu(p_ref[0].astype(_F32) + p_ref[1].astype(_F32))


def _combine(p):
    return pl.pallas_call(
        _combine_body,
        grid=(NB,),
        in_specs=[pl.BlockSpec((2, BN, H), lambda i: (0, i, 0))],
        out_specs=pl.BlockSpec((BN, H), lambda i: (i, 0)),
        out_shape=jax.ShapeDtypeStruct((NP, H), _F32),
    )(p)


def _edge_agg_body(b, gidx_hbm, dst_hbm, hw_hbm, zeros_hbm, out_hbm,
                   idx_v, dst_v, rows, agg, *sems):
    gsem, ssem = sems[:RBUF], sems[RBUF:]
    cid = lax.axis_index("c")
    sid = lax.axis_index("s")
    my_rows = pl.multiple_of(sid * ROWS_PER_SUB, 8)

    pltpu.sync_copy(zeros_hbm, agg.at[pl.ds(my_rows, ROWS_PER_SUB)])

    def run(base_chunk, k):
        pltpu.sync_copy(gidx_hbm.at[pl.ds(base_chunk, k)],
                        idx_v.at[pl.ds(0, k)])
        pltpu.sync_copy(dst_hbm.at[pl.ds(base_chunk, k)],
                        dst_v.at[pl.ds(0, k)])
        plsc.subcore_barrier()
        for c in range(DEPTH):
            pltpu.async_copy(hw_hbm.at[idx_v.at[c]], rows.at[c % RBUF],
                             gsem[c % RBUF])

        def group(jj, carry):
            for kk in range(RBUF):
                c = jj * RBUF + kk

                @pl.when(c < k)
                def _():
                    pltpu.make_async_copy(hw_hbm.at[idx_v.at[c]],
                                          rows.at[kk], gsem[kk]).wait()
                    pltpu.async_copy(rows.at[kk], agg.at[dst_v.at[c]],
                                     ssem[kk], add=True)
                    n = c + DEPTH
                    kn = (kk + DEPTH) % RBUF

                    @pl.when(n < k)
                    def _():
                        @pl.when(n >= RBUF)
                        def _():
                            pltpu.make_async_copy(
                                rows.at[kn], agg.at[dst_v.at[n - RBUF]],
                                ssem[kn]).wait()
                        pltpu.async_copy(hw_hbm.at[idx_v.at[n]],
                                         rows.at[kn], gsem[kn])
            return carry

        lax.fori_loop(0, (k + RBUF - 1) // RBUF, group, 0)
        for c in range(k - RBUF, k):
            pltpu.make_async_copy(rows.at[c % RBUF], agg.at[dst_v.at[c]],
                                  ssem[c % RBUF]).wait()

    bbase = b * NSUB * CHUNKS_SID

    @pl.when(cid == 0)
    def _():
        run(bbase + sid * CHUNKS_SID, K0)

    @pl.when(cid == 1)
    def _():
        run(bbase + sid * CHUNKS_SID + K0, K1)

    plsc.subcore_barrier()
    off = pl.multiple_of(cid * NP + sid * ROWS_PER_SUB, 8)
    pltpu.sync_copy(agg.at[pl.ds(my_rows, ROWS_PER_SUB)],
                    out_hbm.at[pl.ds(off, ROWS_PER_SUB)])


def _edge_agg(gidx, dst, hw_flat, zeros, b):
    import functools
    mesh = plsc.VectorSubcoreMesh(core_axis_name="c", subcore_axis_name="s",
                                  num_cores=NCORES, num_subcores=NSUB)
    k = pl.kernel(
        functools.partial(_edge_agg_body, b),
        out_type=jax.ShapeDtypeStruct((NCORES * NP, H), _BF16),
        mesh=mesh,
        scratch_types=[
            pltpu.VMEM((K0, C), jnp.int32),
            pltpu.VMEM((K0, C), jnp.int32),
            pltpu.VMEM((RBUF, C, H), _BF16),
            pltpu.VMEM_SHARED((NP, H), _BF16),
        ] + [pltpu.SemaphoreType.DMA] * (2 * RBUF),
        compiler_params=pltpu.CompilerParams(use_tc_tiling_on_sc=False),
    )
    return k(gidx, dst, hw_flat, zeros)


def kernel(class_objects, states_objects, edge_tuples, edge_classes,
           mask_object, mask_edge, class_emb, state_W, state_b,
           W1, b1, W2, b2, rgcn_W0, rgcn_W1):
    src = edge_tuples[:, :, 0].astype(jnp.int32)
    dst = edge_tuples[:, :, 1].astype(jnp.int32)
    rel = edge_classes.astype(jnp.int32)
    gidx = ((src // BN) * NR + rel) * BN + src % BN
    gidx_pad = jnp.concatenate(
        [gidx, jnp.zeros((B, EPB - E), jnp.int32)], axis=1)
    dst_pad = jnp.concatenate(
        [dst, jnp.full((B, EPB - E), DUMMY_DST, jnp.int32)], axis=1)
    gidx_pad = gidx_pad.reshape(B * NSUB * CHUNKS_SID, C)
    dst_pad = dst_pad.reshape(B * NSUB * CHUNKS_SID, C)

    cls_pad = jnp.pad(class_objects.astype(_F32), ((0, 0), (0, NP - N)))
    dep = (gidx_pad[0, 0] * 0 + dst_pad[0, 0] * 0).astype(_F32)
    cls_pad = (cls_pad + dep)[:, :, None]
    states_pad = jnp.pad(states_objects, ((0, 0), (0, NP - N), (0, 0)))

    swt = state_W.T
    sb = state_b[None, :]
    w1t, w2t = W1.T, W2.T
    b1r, b2r = b1[None, :], b2[None, :]
    wcat0 = rgcn_W0.transpose(1, 0, 2).reshape(H, NR * H)
    wcat1 = rgcn_W1.transpose(1, 0, 2).reshape(H, NR * H)
    zeros = jnp.zeros((ROWS_PER_SUB, H), _BF16)

    outs = []
    for b in range(B):
        hw1 = _mlp_einsum(cls_pad[b], states_pad[b], class_emb, swt, sb,
                          w1t, b1r, w2t, b2r, wcat0)
        p1 = _edge_agg(gidx_pad, dst_pad, hw1, zeros, b)
        hw2 = _combine_einsum(p1.reshape(NCORES, NP, H), wcat1)
        p2 = _edge_agg(gidx_pad, dst_pad, hw2, zeros, b)
        outs.append(_combine(p2.reshape(NCORES, NP, H)))
    return jnp.stack(outs, axis=0)[:, :N, :]


# speedup vs baseline: 1.0068x; 1.0068x over previous
import jax
import jax.numpy as jnp
from jax import lax
from jax.experimental import pallas as pl
from jax.experimental.pallas import tpu as pltpu
from jax.experimental.pallas import tpu_sc as plsc

B = 4
N = 12500
NP = 12800
E = 200000
EPB = 204800
H = 64
H2 = 32
NS = 30
NC = 300
NR = 16

BN = 1280
NB = NP // BN

NCORES = 2
NSUB = 16
C = 128
CHUNKS_SID = 100
K0 = 78
K1 = CHUNKS_SID - K0
ROWS_PER_SUB = NP // NSUB
DUMMY_DST = 12600

RBUF = 10
DEPTH = 5

_F32 = jnp.float32
_BF16 = jnp.bfloat16


def _mlp_einsum_body(cls_ref, st_ref, cemb_ref, swt_ref, sb_ref,
                     w1t_ref, b1_ref, w2t_ref, b2_ref, wcat_ref, out_ref):
    cls = cls_ref[:, :].astype(jnp.int32)
    iota = lax.broadcasted_iota(jnp.int32, (BN, NC), 1)
    onehot = (iota == cls).astype(_F32)
    cn = jnp.dot(onehot, cemb_ref[...], preferred_element_type=_F32)
    se = jnp.dot(st_ref[...], swt_ref[...], preferred_element_type=_F32)
    se = se + sb_ref[...]
    h = jax.nn.relu(jnp.concatenate([cn, se], axis=1))
    h = jax.nn.relu(jnp.dot(h, w1t_ref[...], preferred_element_type=_F32)
                    + b1_ref[...])
    h = jax.nn.relu(jnp.dot(h, w2t_ref[...], preferred_element_type=_F32)
                    + b2_ref[...])
    hw = jnp.dot(h, wcat_ref[...],
                 preferred_element_type=_F32).astype(_BF16)
    for r in range(NR):
        out_ref[r * BN:(r + 1) * BN, :] = hw[:, r * H:(r + 1) * H]


def _mlp_einsum(cls_b, states_b, class_emb, swt, sb, w1t, b1, w2t, b2, wcat):
    full = lambda shape: pl.BlockSpec(shape, lambda i: (0,) * len(shape))
    return pl.pallas_call(
        _mlp_einsum_body,
        grid=(NB,),
        in_specs=[
            pl.BlockSpec((BN, 1), lambda i: (i, 0)),
            pl.BlockSpec((BN, NS), lambda i: (i, 0)),
            full((NC, H2)), full((NS, H2)), full((1, H2)),
            full((H, H)), full((1, H)), full((H, H)), full((1, H)),
            full((H, NR * H)),
        ],
        out_specs=pl.BlockSpec((NR * BN, H), lambda i: (i, 0)),
        out_shape=jax.ShapeDtypeStruct((NP * NR, H), _BF16),
    )(cls_b, states_b, class_emb, swt, sb, w1t, b1, w2t, b2, wcat)


def _combine_einsum_body(p_ref, wcat_ref, out_ref):
    h = jax.nn.relu(p_ref[0].astype(_F32) + p_ref[1].astype(_F32))
    hw = jnp.dot(h, wcat_ref[...],
                 preferred_element_type=_F32).astype(_BF16)
    for r in range(NR):
        out_ref[r * BN:(r + 1) * BN, :] = hw[:, r * H:(r + 1) * H]


def _combine_einsum(p, wcat):
    return pl.pallas_call(
        _combine_einsum_body,
        grid=(NB,),
        in_specs=[
            pl.BlockSpec((2, BN, H), lambda i: (0, i, 0)),
            pl.BlockSpec((H, NR * H), lambda i: (0, 0)),
        ],
        out_specs=pl.BlockSpec((NR * BN, H), lambda i: (i, 0)),
        out_shape=jax.ShapeDtypeStruct((NP * NR, H), _BF16),
    )(p, wcat)


def _combine_body(p_ref, out_ref):
    out_ref[...] = jax.nn.relu(p_ref[0].astype(_F32) + p_ref[1].astype(_F32))


def _combine(p):
    return pl.pallas_call(
        _combine_body,
        grid=(NB,),
        in_specs=[pl.BlockSpec((2, BN, H), lambda i: (0, i, 0))],
        out_specs=pl.BlockSpec((BN, H), lambda i: (i, 0)),
        out_shape=jax.ShapeDtypeStruct((NP, H), _F32),
    )(p)


def _edge_agg_body(b, gidx_hbm, dst_hbm, hw_hbm, zeros_hbm, out_hbm,
                   idx_v, dst_v, rows, agg, *sems):
    gsem, ssem = sems[:RBUF], sems[RBUF:]
    cid = lax.axis_index("c")
    sid = lax.axis_index("s")
    my_rows = pl.multiple_of(sid * ROWS_PER_SUB, 8)

    pltpu.sync_copy(zeros_hbm, agg.at[pl.ds(my_rows, ROWS_PER_SUB)])

    def run(base_chunk, k):
        pltpu.sync_copy(gidx_hbm.at[pl.ds(base_chunk, k)],
                        idx_v.at[pl.ds(0, k)])
        pltpu.sync_copy(dst_hbm.at[pl.ds(base_chunk, k)],
                        dst_v.at[pl.ds(0, k)])
        plsc.subcore_barrier()
        for c in range(DEPTH):
            pltpu.async_copy(hw_hbm.at[idx_v.at[c]], rows.at[c % RBUF],
                             gsem[c % RBUF])

        def group(jj, carry):
            for kk in range(RBUF):
                c = jj * RBUF + kk

                @pl.when(c < k)
                def _():
                    pltpu.make_async_copy(hw_hbm.at[idx_v.at[c]],
                                          rows.at[kk], gsem[kk]).wait()
                    pltpu.async_copy(rows.at[kk], agg.at[dst_v.at[c]],
                                     ssem[kk], add=True)
                    n = c + DEPTH
                    kn = (kk + DEPTH) % RBUF

                    @pl.when(n < k)
                    def _():
                        @pl.when(n >= RBUF)
                        def _():
                            pltpu.make_async_copy(
                                rows.at[kn], agg.at[dst_v.at[n - RBUF]],
                                ssem[kn]).wait()
                        pltpu.async_copy(hw_hbm.at[idx_v.at[n]],
                                         rows.at[kn], gsem[kn])
            return carry

        lax.fori_loop(0, (k + RBUF - 1) // RBUF, group, 0)
        for c in range(k - RBUF, k):
            pltpu.make_async_copy(rows.at[c % RBUF], agg.at[dst_v.at[c]],
                                  ssem[c % RBUF]).wait()

    bbase = b * NSUB * CHUNKS_SID

    @pl.when(cid == 0)
    def _():
        run(bbase + sid * CHUNKS_SID, K0)

    @pl.when(cid == 1)
    def _():
        run(bbase + sid * CHUNKS_SID + K0, K1)

    plsc.subcore_barrier()
    off = pl.multiple_of(cid * NP + sid * ROWS_PER_SUB, 8)
    pltpu.sync_copy(agg.at[pl.ds(my_rows, ROWS_PER_SUB)],
                    out_hbm.at[pl.ds(off, ROWS_PER_SUB)])


def _edge_agg(gidx, dst, hw_flat, zeros, b):
    import functools
    mesh = plsc.VectorSubcoreMesh(core_axis_name="c", subcore_axis_name="s",
                                  num_cores=NCORES, num_subcores=NSUB)
    k = pl.kernel(
        functools.partial(_edge_agg_body, b),
        out_type=jax.ShapeDtypeStruct((NCORES * NP, H), _BF16),
        mesh=mesh,
        scratch_types=[
            pltpu.VMEM((K0, C), jnp.int32),
            pltpu.VMEM((K0, C), jnp.int32),
            pltpu.VMEM((RBUF, C, H), _BF16),
            pltpu.VMEM_SHARED((NP, H), _BF16),
        ] + [pltpu.SemaphoreType.DMA] * (2 * RBUF),
        compiler_params=pltpu.CompilerParams(use_tc_tiling_on_sc=False),
    )
    return k(gidx, dst, hw_flat, zeros)


def kernel(class_objects, states_objects, edge_tuples, edge_classes,
           mask_object, mask_edge, class_emb, state_W, state_b,
           W1, b1, W2, b2, rgcn_W0, rgcn_W1):
    src = edge_tuples[:, :, 0].astype(jnp.int32)
    dst = edge_tuples[:, :, 1].astype(jnp.int32)
    rel = edge_classes.astype(jnp.int32)
    gidx = ((src // BN) * NR + rel) * BN + src % BN
    gidx_pad = jnp.concatenate(
        [gidx, jnp.zeros((B, EPB - E), jnp.int32)], axis=1)
    dst_pad = jnp.concatenate(
        [dst, jnp.full((B, EPB - E), DUMMY_DST, jnp.int32)], axis=1)
    gidx_pad = gidx_pad.reshape(B * NSUB * CHUNKS_SID, C)
    dst_pad = dst_pad.reshape(B * NSUB * CHUNKS_SID, C)

    cls_pad = jnp.pad(class_objects.astype(_F32), ((0, 0), (0, NP - N)))
    dep = (gidx_pad[0, 0] * 0 + dst_pad[0, 0] * 0).astype(_F32)
    cls_pad = (cls_pad + dep)[:, :, None]
    states_pad = jnp.pad(states_objects, ((0, 0), (0, NP - N), (0, 0)))

    swt = state_W.T
    sb = state_b[None, :]
    w1t, w2t = W1.T, W2.T
    b1r, b2r = b1[None, :], b2[None, :]
    wcat0 = rgcn_W0.transpose(1, 0, 2).reshape(H, NR * H)
    wcat1 = rgcn_W1.transpose(1, 0, 2).reshape(H, NR * H)
    zeros = jnp.zeros((ROWS_PER_SUB, H), _BF16)

    outs = []
    for b in range(B):
        hw1 = _mlp_einsum(cls_pad[b], states_pad[b], class_emb, swt, sb,
                          w1t, b1r, w2t, b2r, wcat0)
        p1 = _edge_agg(gidx_pad, dst_pad, hw1, zeros, b)
        hw2 = _combine_einsum(p1.reshape(NCORES, NP, H), wcat1)
        p2 = _edge_agg(gidx_pad, dst_pad, hw2, zeros, b)
        outs.append(_combine(p2.reshape(NCORES, NP, H)))
    return jnp.stack(outs, axis=0)[:, :N, :]
